# Initial kernel scaffold; baseline (speedup 1.0000x reference)
import jax
import jax.numpy as jnp
from jax.experimental import pallas as pl


def kernel(x, edge_index, edge_attr, w0, root0, b0, w1, root1, b1, w2, root2, b2, w3, root3, b3, w4, root4, b4, w5, root5, b5, lin_w, lin_b):
    raise NotImplementedError("write your pallas kernel here")


# speedup vs baseline: 3.4944x; 3.4944x over previous
import functools

import jax
import jax.numpy as jnp
from jax import lax
from jax.experimental import pallas as pl
from jax.experimental.pallas import tpu as pltpu
from jax.experimental.pallas import tpu_sc as plsc

_KS = 5
_DIM = 3
_K = _KS ** _DIM
_N = 2048
_E = 8192
_F = 64
_IN0 = 9
_TAPS = 2 ** _DIM
_T = _E * _TAPS
_NC = 2
_NS = 16
_NW = _NC * _NS
_TPW = _T // _NW
_C = 128
_NCH = _TPW // _C
_ROWS_PER_SUB = _N // _NS
_FP = 128


def _prep_body(attr_t_ref, ei_ref, keys_ref, dstv_ref, bas_ref, invdeg_ref):
    src = ei_ref[0, :]
    dst = ei_ref[1, :]
    p = [attr_t_ref[d, :] * (_KS - 1.0) for d in range(_DIM)]
    lo = [jnp.floor(p[d]) for d in range(_DIM)]
    frac = [p[d] - lo[d] for d in range(_DIM)]
    lo_i = [jnp.clip(lo[d].astype(jnp.int32), 0, _KS - 1) for d in range(_DIM)]
    for b in range(_TAPS):
        basis = jnp.ones((_E,), jnp.float32)
        widx = jnp.zeros((_E,), jnp.int32)
        for d in range(_DIM):
            bit = (b >> d) & 1
            basis = basis * (frac[d] if bit else (1.0 - frac[d]))
            ii = jnp.clip(lo_i[d] + bit, 0, _KS - 1)
            widx = widx * _KS + ii
        keys_ref[b, :] = widx * _N + src
        dstv_ref[b, :] = dst
        bas_ref[b, :] = basis
    acc = jnp.zeros((_N, 1), jnp.float32)
    iota_n = lax.broadcasted_iota(jnp.int32, (_N, 512), 0)
    for c in range(_E // 512):
        dst_c = dst[c * 512:(c + 1) * 512]
        m = (iota_n == dst_c[None, :]).astype(jnp.float32)
        acc = acc + jnp.sum(m, axis=1, keepdims=True)
    invdeg_ref[...] = 1.0 / jnp.maximum(acc, 1.0)


def _prep(attr_t, edge_index):
    return pl.pallas_call(
        _prep_body,
        out_shape=(
            jax.ShapeDtypeStruct((_TAPS, _E), jnp.int32),
            jax.ShapeDtypeStruct((_TAPS, _E), jnp.int32),
            jax.ShapeDtypeStruct((_TAPS, _E), jnp.float32),
            jax.ShapeDtypeStruct((_N, 1), jnp.float32),
        ),
    )(attr_t, edge_index)


def _mm0_body(x_ref, w_ref, r_ref, hall_ref, root_ref):
    k = pl.program_id(0)
    x = x_ref[...]
    hall_ref[0] = jnp.dot(x, w_ref[0], preferred_element_type=jnp.float32)

    @pl.when(k == 0)
    def _():
        root_ref[...] = jnp.dot(x, r_ref[...], preferred_element_type=jnp.float32)


def _mm0(x, w, r):
    in_ch = x.shape[1]
    return pl.pallas_call(
        _mm0_body,
        grid=(_K,),
        in_specs=[
            pl.BlockSpec((_N, in_ch), lambda k: (0, 0)),
            pl.BlockSpec((1, in_ch, _FP), lambda k: (k, 0, 0)),
            pl.BlockSpec((in_ch, _F), lambda k: (0, 0)),
        ],
        out_specs=[
            pl.BlockSpec((1, _N, _FP), lambda k: (k, 0, 0)),
            pl.BlockSpec((_N, _F), lambda k: (0, 0)),
        ],
        out_shape=(
            jax.ShapeDtypeStruct((_K, _N, _FP), jnp.float32),
            jax.ShapeDtypeStruct((_N, _F), jnp.float32),
        ),
    )(x, w, r)


def _mm_mid_body(parts_ref, root_ref, b_ref, invdeg_ref, w_ref, r_ref,
                 hall_ref, rootout_ref, hout_ref, h_scr):
    k = pl.program_id(0)

    @pl.when(k == 0)
    def _():
        msg = (parts_ref[0, :, 0:_F] + parts_ref[1, :, 0:_F]) * invdeg_ref[...]
        h = jnp.maximum(msg + root_ref[...] + b_ref[...], 0.0)
        h_scr[...] = h
        hout_ref[...] = h
        rootout_ref[...] = jnp.dot(h, r_ref[...], preferred_element_type=jnp.float32)

    hall_ref[0] = jnp.dot(h_scr[...], w_ref[0], preferred_element_type=jnp.float32)


def _mm_mid(parts, root, b, invdeg, w, r):
    return pl.pallas_call(
        _mm_mid_body,
        grid=(_K,),
        in_specs=[
            pl.BlockSpec((2, _N, _FP), lambda k: (0, 0, 0)),
            pl.BlockSpec((_N, _F), lambda k: (0, 0)),
            pl.BlockSpec((1, _F), lambda k: (0, 0)),
            pl.BlockSpec((_N, 1), lambda k: (0, 0)),
            pl.BlockSpec((1, _F, _FP), lambda k: (k, 0, 0)),
            pl.BlockSpec((_F, _F), lambda k: (0, 0)),
        ],
        out_specs=[
            pl.BlockSpec((1, _N, _FP), lambda k: (k, 0, 0)),
            pl.BlockSpec((_N, _F), lambda k: (0, 0)),
            pl.BlockSpec((_N, _F), lambda k: (0, 0)),
        ],
        out_shape=(
            jax.ShapeDtypeStruct((_K, _N, _FP), jnp.float32),
            jax.ShapeDtypeStruct((_N, _F), jnp.float32),
            jax.ShapeDtypeStruct((_N, _F), jnp.float32),
        ),
        scratch_shapes=[pltpu.VMEM((_N, _F), jnp.float32)],
    )(parts, root, b, invdeg, w, r)


def _final_body(parts_ref, root_ref, b_ref, invdeg_ref, x_ref,
                h1, h2, h3, h4, h5, linw_ref, linb_ref, out_ref):
    msg = (parts_ref[0, :, 0:_F] + parts_ref[1, :, 0:_F]) * invdeg_ref[...]
    h6 = jnp.maximum(msg + root_ref[...] + b_ref[...], 0.0)
    dn = (((0,), (1,)), ((), ()))
    acc = lax.dot_general(linw_ref[pl.ds(0, _IN0), :], x_ref[...], dn,
                          preferred_element_type=jnp.float32)
    feats = [h1[...], h2[...], h3[...], h4[...], h5[...], h6]
    for i, f in enumerate(feats):
        wslice = linw_ref[pl.ds(_IN0 + i * _F, _F), :]
        acc = acc + lax.dot_general(wslice, f, dn, preferred_element_type=jnp.float32)
    out_ref[...] = acc + linb_ref[...]


def _final(parts, root, b, invdeg, x, hs, lin_w, lin_b):
    return pl.pallas_call(
        _final_body,
        out_shape=jax.ShapeDtypeStruct((_F, _N), jnp.float32),
    )(parts, root, b, invdeg, x, *hs, lin_w, lin_b)


_sc_mesh = plsc.VectorSubcoreMesh(core_axis_name="c", subcore_axis_name="s",
                                  num_cores=_NC, num_subcores=_NS)


@functools.partial(
    pl.kernel,
    out_type=jax.ShapeDtypeStruct((_NC, _N, _FP), jnp.float32),
    mesh=_sc_mesh,
    scratch_types=[
        pltpu.VMEM((_C,), jnp.int32),
        pltpu.VMEM((_C,), jnp.int32),
        pltpu.VMEM((_C,), jnp.float32),
        pltpu.VMEM((_C, _FP), jnp.float32),
        pltpu.VMEM((_ROWS_PER_SUB, _FP), jnp.float32),
        pltpu.VMEM_SHARED((_N, _FP), jnp.float32),
        pltpu.SemaphoreType.DMA,
    ],
)
def _sc_scatter(hall, keys, dstv, bas, out, key_v, dst_v, bas_v, rows_v,
                zero_v, acc_sh, sem):
    c = lax.axis_index("c")
    s = lax.axis_index("s")
    base = (c * _NS + s) * _TPW

    def _zrow(r, _):
        for q in range(_FP // 16):
            zero_v[r, pl.ds(q * 16, 16)] = jnp.zeros((16,), jnp.float32)
        return 0

    lax.fori_loop(0, _ROWS_PER_SUB, _zrow, 0)
    pltpu.sync_copy(zero_v, acc_sh.at[pl.ds(s * _ROWS_PER_SUB, _ROWS_PER_SUB)])
    plsc.subcore_barrier()

    for i in range(_NCH):
        off = base + i * _C
        pltpu.sync_copy(keys.at[pl.ds(off, _C)], key_v)
        pltpu.sync_copy(dstv.at[pl.ds(off, _C)], dst_v)
        pltpu.sync_copy(bas.at[pl.ds(off, _C)], bas_v)
        pltpu.async_copy(hall.at[key_v], rows_v, sem).wait()

        def _scale(jb, _):
            bchunk = bas_v[pl.ds(jb * 16, 16)]
            for t in range(16):
                bv = bchunk[t]
                r = jb * 16 + t
                for q in range(_F // 16):
                    rows_v[r, pl.ds(q * 16, 16)] = rows_v[r, pl.ds(q * 16, 16)] * bv
            return 0

        lax.fori_loop(0, _C // 16, _scale, 0)
        pltpu.sync_copy(rows_v, acc_sh.at[dst_v], add=True)

    plsc.subcore_barrier()
    pltpu.sync_copy(
        acc_sh.at[pl.ds(s * _ROWS_PER_SUB, _ROWS_PER_SUB)],
        out.at[c, pl.ds(s * _ROWS_PER_SUB, _ROWS_PER_SUB)],
    )


def kernel(x, edge_index, edge_attr, w0, root0, b0, w1, root1, b1, w2, root2,
           b2, w3, root3, b3, w4, root4, b4, w5, root5, b5, lin_w, lin_b):
    ws = [w0, w1, w2, w3, w4, w5]
    rs = [root0, root1, root2, root3, root4, root5]
    bs = [b.reshape(1, _F) for b in [b0, b1, b2, b3, b4, b5]]

    keys, dstv, bas, invdeg = _prep(edge_attr.T, edge_index)
    keys_f = keys.reshape(_T)
    dstv_f = dstv.reshape(_T)
    bas_f = bas.reshape(_T)

    wsp = [jnp.pad(w, ((0, 0), (0, 0), (0, _FP - _F))) for w in ws]
    hall, root = _mm0(x, wsp[0], rs[0])
    hs = []
    for l in range(1, 6):
        parts = _sc_scatter(hall.reshape(_K * _N, _FP), keys_f, dstv_f, bas_f)
        hall, root, h = _mm_mid(parts, root, bs[l - 1], invdeg, wsp[l], rs[l])
        hs.append(h)
    parts = _sc_scatter(hall.reshape(_K * _N, _FP), keys_f, dstv_f, bas_f)
    return _final(parts, root, bs[5], invdeg, x, hs, lin_w,
                  lin_b.reshape(_F, 1))


# speedup vs baseline: 4.2873x; 1.2269x over previous
import functools

import jax
import jax.numpy as jnp
from jax import lax
from jax.experimental import pallas as pl
from jax.experimental.pallas import tpu as pltpu
from jax.experimental.pallas import tpu_sc as plsc

_KS = 5
_DIM = 3
_K = _KS ** _DIM
_N = 2048
_E = 8192
_F = 64
_IN0 = 9
_TAPS = 2 ** _DIM
_T = _E * _TAPS
_NC = 2
_NS = 16
_NW = _NC * _NS
_TPW = _T // _NW
_C = 128
_NCH = _TPW // _C
_ROWS_PER_SUB = _N // _NS
_FP = 128


def _prep_body(attr_t_ref, ei_ref, keys_ref, dstv_ref, bas_ref, invdeg_ref):
    src = ei_ref[0, :]
    dst = ei_ref[1, :]
    p = [attr_t_ref[d, :] * (_KS - 1.0) for d in range(_DIM)]
    lo = [jnp.floor(p[d]) for d in range(_DIM)]
    frac = [p[d] - lo[d] for d in range(_DIM)]
    lo_i = [jnp.clip(lo[d].astype(jnp.int32), 0, _KS - 1) for d in range(_DIM)]
    for b in range(_TAPS):
        basis = jnp.ones((_E,), jnp.float32)
        widx = jnp.zeros((_E,), jnp.int32)
        for d in range(_DIM):
            bit = (b >> d) & 1
            basis = basis * (frac[d] if bit else (1.0 - frac[d]))
            ii = jnp.clip(lo_i[d] + bit, 0, _KS - 1)
            widx = widx * _KS + ii
        keys_ref[b, :] = widx * _N + src
        dstv_ref[b, :] = dst
        bas_ref[b, :] = basis
    acc = jnp.zeros((_N, 1), jnp.float32)
    iota_n = lax.broadcasted_iota(jnp.int32, (_N, 512), 0)
    for c in range(_E // 512):
        dst_c = dst[c * 512:(c + 1) * 512]
        m = (iota_n == dst_c[None, :]).astype(jnp.float32)
        acc = acc + jnp.sum(m, axis=1, keepdims=True)
    invdeg_ref[...] = 1.0 / jnp.maximum(acc, 1.0)


def _prep(attr_t, edge_index):
    return pl.pallas_call(
        _prep_body,
        out_shape=(
            jax.ShapeDtypeStruct((_TAPS, _E), jnp.int32),
            jax.ShapeDtypeStruct((_TAPS, _E), jnp.int32),
            jax.ShapeDtypeStruct((_TAPS, _E), jnp.float32),
            jax.ShapeDtypeStruct((_N, 1), jnp.float32),
        ),
    )(attr_t, edge_index)


def _mm0_body(x_ref, w_ref, r_ref, hall_ref, root_ref):
    k = pl.program_id(0)
    x = x_ref[...]
    hall_ref[0] = jnp.dot(x, w_ref[0], preferred_element_type=jnp.float32)

    @pl.when(k == 0)
    def _():
        root_ref[...] = jnp.dot(x, r_ref[...], preferred_element_type=jnp.float32)


def _mm0(x, w, r):
    in_ch = x.shape[1]
    return pl.pallas_call(
        _mm0_body,
        grid=(_K,),
        in_specs=[
            pl.BlockSpec((_N, in_ch), lambda k: (0, 0)),
            pl.BlockSpec((1, in_ch, _FP), lambda k: (k, 0, 0)),
            pl.BlockSpec((in_ch, _F), lambda k: (0, 0)),
        ],
        out_specs=[
            pl.BlockSpec((1, _N, _FP), lambda k: (k, 0, 0)),
            pl.BlockSpec((_N, _F), lambda k: (0, 0)),
        ],
        out_shape=(
            jax.ShapeDtypeStruct((_K, _N, _FP), jnp.float32),
            jax.ShapeDtypeStruct((_N, _F), jnp.float32),
        ),
    )(x, w, r)


def _mm_mid_body(parts_ref, root_ref, b_ref, invdeg_ref, w_ref, r_ref,
                 hall_ref, rootout_ref, hout_ref, h_scr):
    k = pl.program_id(0)

    @pl.when(k == 0)
    def _():
        msg = (parts_ref[0, :, 0:_F] + parts_ref[1, :, 0:_F]) * invdeg_ref[...]
        h = jnp.maximum(msg + root_ref[...] + b_ref[...], 0.0)
        h_scr[...] = h
        hout_ref[...] = h
        rootout_ref[...] = jnp.dot(h, r_ref[...], preferred_element_type=jnp.float32)

    hall_ref[0] = jnp.dot(h_scr[...], w_ref[0], preferred_element_type=jnp.float32)


def _mm_mid(parts, root, b, invdeg, w, r):
    return pl.pallas_call(
        _mm_mid_body,
        grid=(_K,),
        in_specs=[
            pl.BlockSpec((2, _N, _FP), lambda k: (0, 0, 0)),
            pl.BlockSpec((_N, _F), lambda k: (0, 0)),
            pl.BlockSpec((1, _F), lambda k: (0, 0)),
            pl.BlockSpec((_N, 1), lambda k: (0, 0)),
            pl.BlockSpec((1, _F, _FP), lambda k: (k, 0, 0)),
            pl.BlockSpec((_F, _F), lambda k: (0, 0)),
        ],
        out_specs=[
            pl.BlockSpec((1, _N, _FP), lambda k: (k, 0, 0)),
            pl.BlockSpec((_N, _F), lambda k: (0, 0)),
            pl.BlockSpec((_N, _F), lambda k: (0, 0)),
        ],
        out_shape=(
            jax.ShapeDtypeStruct((_K, _N, _FP), jnp.float32),
            jax.ShapeDtypeStruct((_N, _F), jnp.float32),
            jax.ShapeDtypeStruct((_N, _F), jnp.float32),
        ),
        scratch_shapes=[pltpu.VMEM((_N, _F), jnp.float32)],
    )(parts, root, b, invdeg, w, r)


def _final_body(parts_ref, root_ref, b_ref, invdeg_ref, x_ref,
                h1, h2, h3, h4, h5, linw_ref, linb_ref, out_ref):
    msg = (parts_ref[0, :, 0:_F] + parts_ref[1, :, 0:_F]) * invdeg_ref[...]
    h6 = jnp.maximum(msg + root_ref[...] + b_ref[...], 0.0)
    dn = (((0,), (1,)), ((), ()))
    acc = lax.dot_general(linw_ref[pl.ds(0, _IN0), :], x_ref[...], dn,
                          preferred_element_type=jnp.float32)
    feats = [h1[...], h2[...], h3[...], h4[...], h5[...], h6]
    for i, f in enumerate(feats):
        wslice = linw_ref[pl.ds(_IN0 + i * _F, _F), :]
        acc = acc + lax.dot_general(wslice, f, dn, preferred_element_type=jnp.float32)
    out_ref[...] = acc + linb_ref[...]


def _final(parts, root, b, invdeg, x, hs, lin_w, lin_b):
    return pl.pallas_call(
        _final_body,
        out_shape=jax.ShapeDtypeStruct((_F, _N), jnp.float32),
    )(parts, root, b, invdeg, x, *hs, lin_w, lin_b)


_sc_mesh = plsc.VectorSubcoreMesh(core_axis_name="c", subcore_axis_name="s",
                                  num_cores=_NC, num_subcores=_NS)


@functools.partial(
    pl.kernel,
    out_type=jax.ShapeDtypeStruct((_NC, _N, _FP), jnp.float32),
    mesh=_sc_mesh,
    scratch_types=[
        pltpu.VMEM((_NCH, _C), jnp.int32),
        pltpu.VMEM((_NCH, _C), jnp.int32),
        pltpu.VMEM((_NCH, _C), jnp.float32),
        pltpu.VMEM((_C, _FP), jnp.float32),
        pltpu.VMEM((_C, _FP), jnp.float32),
        pltpu.VMEM((_ROWS_PER_SUB, _FP), jnp.float32),
        pltpu.VMEM_SHARED((_N, _FP), jnp.float32),
        pltpu.SemaphoreType.DMA,
        pltpu.SemaphoreType.DMA,
        pltpu.SemaphoreType.DMA,
    ],
)
def _sc_scatter(hall, keys2, dstv2, bas2, out, key_v, dst_v, bas_v, rows0,
                rows1, zero_v, acc_sh, sem0, sem1, semi):
    c = lax.axis_index("c")
    s = lax.axis_index("s")
    w = c * _NS + s
    cbase = w * _NCH

    di1 = pltpu.async_copy(keys2.at[pl.ds(cbase, _NCH)], key_v, semi)
    di2 = pltpu.async_copy(dstv2.at[pl.ds(cbase, _NCH)], dst_v, semi)
    di3 = pltpu.async_copy(bas2.at[pl.ds(cbase, _NCH)], bas_v, semi)

    @plsc.parallel_loop(0, _ROWS_PER_SUB)
    def _zrow(r):
        for q in range(_FP // 16):
            zero_v[r, pl.ds(q * 16, 16)] = jnp.zeros((16,), jnp.float32)

    pltpu.sync_copy(zero_v, acc_sh.at[pl.ds(s * _ROWS_PER_SUB, _ROWS_PER_SUB)])
    di1.wait()
    di2.wait()
    di3.wait()
    plsc.subcore_barrier()

    bufs = (rows0, rows1)
    sems = (sem0, sem1)
    descs = [None, None]
    descs[0] = pltpu.async_copy(hall.at[key_v.at[0]], rows0, sem0)
    for i in range(_NCH):
        b = i % 2
        descs[b].wait()
        if i + 1 < _NCH:
            nb = (i + 1) % 2
            descs[nb] = pltpu.async_copy(hall.at[key_v.at[i + 1]], bufs[nb],
                                         sems[nb])
        rows_v = bufs[b]

        @plsc.parallel_loop(0, _C // 16)
        def _scale(jb):
            bchunk = bas_v[i, pl.ds(jb * 16, 16)]
            for t in range(16):
                bv = bchunk[t]
                r = jb * 16 + t
                for q in range(_F // 16):
                    rows_v[r, pl.ds(q * 16, 16)] = rows_v[r, pl.ds(q * 16, 16)] * bv

        pltpu.sync_copy(rows_v, acc_sh.at[dst_v.at[i]], add=True)

    plsc.subcore_barrier()
    pltpu.sync_copy(
        acc_sh.at[pl.ds(s * _ROWS_PER_SUB, _ROWS_PER_SUB)],
        out.at[c, pl.ds(s * _ROWS_PER_SUB, _ROWS_PER_SUB)],
    )


def kernel(x, edge_index, edge_attr, w0, root0, b0, w1, root1, b1, w2, root2,
           b2, w3, root3, b3, w4, root4, b4, w5, root5, b5, lin_w, lin_b):
    ws = [w0, w1, w2, w3, w4, w5]
    rs = [root0, root1, root2, root3, root4, root5]
    bs = [b.reshape(1, _F) for b in [b0, b1, b2, b3, b4, b5]]

    keys, dstv, bas, invdeg = _prep(edge_attr.T, edge_index)
    keys_f = keys.reshape(_T // _C, _C)
    dstv_f = dstv.reshape(_T // _C, _C)
    bas_f = bas.reshape(_T // _C, _C)

    wsp = [jnp.pad(w, ((0, 0), (0, 0), (0, _FP - _F))) for w in ws]
    hall, root = _mm0(x, wsp[0], rs[0])
    hs = []
    for l in range(1, 6):
        parts = _sc_scatter(hall.reshape(_K * _N, _FP), keys_f, dstv_f, bas_f)
        hall, root, h = _mm_mid(parts, root, bs[l - 1], invdeg, wsp[l], rs[l])
        hs.append(h)
    parts = _sc_scatter(hall.reshape(_K * _N, _FP), keys_f, dstv_f, bas_f)
    return _final(parts, root, bs[5], invdeg, x, hs, lin_w,
                  lin_b.reshape(_F, 1))


# speedup vs baseline: 6.5225x; 1.5213x over previous
import functools

import jax
import jax.numpy as jnp
from jax import lax
from jax.experimental import pallas as pl
from jax.experimental.pallas import tpu as pltpu
from jax.experimental.pallas import tpu_sc as plsc

_KS = 5
_DIM = 3
_K = _KS ** _DIM
_N = 2048
_E = 8192
_F = 64
_IN0 = 9
_TAPS = 2 ** _DIM
_T = _E * _TAPS
_NC = 2
_NS = 16
_NW = _NC * _NS
_TPW = _T // _NW
_C = 128
_NCH = _TPW // _C
_ROWS_PER_SUB = _N // _NS
_FP = 128
_KP = _K + 1
_KT = _KP // 2


def _prep_body(attr_t_ref, ei_ref, keys_ref, dstv_ref, blo_ref, bhi_ref,
               invdeg_ref):
    src = ei_ref[0, :]
    dst = ei_ref[1, :]
    p = [attr_t_ref[d, :] * (_KS - 1.0) for d in range(_DIM)]
    lo = [jnp.floor(p[d]) for d in range(_DIM)]
    frac = [p[d] - lo[d] for d in range(_DIM)]
    lo_i = [jnp.clip(lo[d].astype(jnp.int32), 0, _KS - 1) for d in range(_DIM)]
    for b in range(_TAPS):
        basis = jnp.ones((_E,), jnp.float32)
        widx = jnp.zeros((_E,), jnp.int32)
        for d in range(_DIM):
            bit = (b >> d) & 1
            basis = basis * (frac[d] if bit else (1.0 - frac[d]))
            ii = jnp.clip(lo_i[d] + bit, 0, _KS - 1)
            widx = widx * _KS + ii
        keys_ref[b, :] = ((src // 8) * _KT + widx // 2) * 8 + (src % 8)
        dstv_ref[b, :] = dst
        even = (widx % 2) == 0
        blo_ref[b, :] = jnp.where(even, basis, 0.0)
        bhi_ref[b, :] = jnp.where(even, 0.0, basis)
    acc = jnp.zeros((_N, 1), jnp.float32)
    iota_n = lax.broadcasted_iota(jnp.int32, (_N, 512), 0)
    for c in range(_E // 512):
        dst_c = dst[c * 512:(c + 1) * 512]
        m = (iota_n == dst_c[None, :]).astype(jnp.float32)
        acc = acc + jnp.sum(m, axis=1, keepdims=True)
    invdeg_ref[...] = 1.0 / jnp.maximum(acc, 1.0)


def _prep(attr_t, edge_index):
    return pl.pallas_call(
        _prep_body,
        out_shape=(
            jax.ShapeDtypeStruct((_TAPS, _E), jnp.int32),
            jax.ShapeDtypeStruct((_TAPS, _E), jnp.int32),
            jax.ShapeDtypeStruct((_TAPS, _E), jnp.float32),
            jax.ShapeDtypeStruct((_TAPS, _E), jnp.float32),
            jax.ShapeDtypeStruct((_N, 1), jnp.float32),
        ),
    )(attr_t, edge_index)


def _mm0_body(x_ref, w_ref, r_ref, hall_ref, root_ref):
    k = pl.program_id(0)
    x = x_ref[...]
    y = jnp.dot(x, w_ref[...], preferred_element_type=jnp.float32)
    hall_ref[:, 0] = y.reshape(_N // 8, 8, _FP)

    @pl.when(k == 0)
    def _():
        root_ref[...] = jnp.dot(x, r_ref[...], preferred_element_type=jnp.float32)


def _mm0(x, wcat, r):
    in_ch = x.shape[1]
    return pl.pallas_call(
        _mm0_body,
        grid=(_KT,),
        in_specs=[
            pl.BlockSpec((_N, in_ch), lambda k: (0, 0)),
            pl.BlockSpec((in_ch, _FP), lambda k: (0, k)),
            pl.BlockSpec((in_ch, _F), lambda k: (0, 0)),
        ],
        out_specs=[
            pl.BlockSpec((_N // 8, 1, 8, _FP), lambda k: (0, k, 0, 0)),
            pl.BlockSpec((_N, _F), lambda k: (0, 0)),
        ],
        out_shape=(
            jax.ShapeDtypeStruct((_N // 8, _KT, 8, _FP), jnp.float32),
            jax.ShapeDtypeStruct((_N, _F), jnp.float32),
        ),
    )(x, wcat, r)


def _mm_mid_body(parts_ref, root_ref, b_ref, invdeg_ref, w_ref, r_ref,
                 hall_ref, rootout_ref, hout_ref, h_scr):
    k = pl.program_id(0)

    @pl.when(k == 0)
    def _():
        msg = (parts_ref[0, :, 0:_F] + parts_ref[0, :, _F:_FP]
               + parts_ref[1, :, 0:_F] + parts_ref[1, :, _F:_FP]) * invdeg_ref[...]
        h = jnp.maximum(msg + root_ref[...] + b_ref[...], 0.0)
        h_scr[...] = h
        hout_ref[...] = h
        rootout_ref[...] = jnp.dot(h, r_ref[...], preferred_element_type=jnp.float32)

    y = jnp.dot(h_scr[...], w_ref[...], preferred_element_type=jnp.float32)
    hall_ref[:, 0] = y.reshape(_N // 8, 8, _FP)


def _mm_mid(parts, root, b, invdeg, w, r):
    return pl.pallas_call(
        _mm_mid_body,
        grid=(_KT,),
        in_specs=[
            pl.BlockSpec((2, _N, _FP), lambda k: (0, 0, 0)),
            pl.BlockSpec((_N, _F), lambda k: (0, 0)),
            pl.BlockSpec((1, _F), lambda k: (0, 0)),
            pl.BlockSpec((_N, 1), lambda k: (0, 0)),
            pl.BlockSpec((_F, _FP), lambda k: (0, k)),
            pl.BlockSpec((_F, _F), lambda k: (0, 0)),
        ],
        out_specs=[
            pl.BlockSpec((_N // 8, 1, 8, _FP), lambda k: (0, k, 0, 0)),
            pl.BlockSpec((_N, _F), lambda k: (0, 0)),
            pl.BlockSpec((_N, _F), lambda k: (0, 0)),
        ],
        out_shape=(
            jax.ShapeDtypeStruct((_N // 8, _KT, 8, _FP), jnp.float32),
            jax.ShapeDtypeStruct((_N, _F), jnp.float32),
            jax.ShapeDtypeStruct((_N, _F), jnp.float32),
        ),
        scratch_shapes=[pltpu.VMEM((_N, _F), jnp.float32)],
    )(parts, root, b, invdeg, w, r)


def _final_body(parts_ref, root_ref, b_ref, invdeg_ref, x_ref,
                h1, h2, h3, h4, h5, linw_ref, linb_ref, out_ref):
    msg = (parts_ref[0, :, 0:_F] + parts_ref[0, :, _F:_FP]
           + parts_ref[1, :, 0:_F] + parts_ref[1, :, _F:_FP]) * invdeg_ref[...]
    h6 = jnp.maximum(msg + root_ref[...] + b_ref[...], 0.0)
    dn = (((0,), (1,)), ((), ()))
    acc = lax.dot_general(linw_ref[pl.ds(0, _IN0), :], x_ref[...], dn,
                          preferred_element_type=jnp.float32)
    feats = [h1[...], h2[...], h3[...], h4[...], h5[...], h6]
    for i, f in enumerate(feats):
        wslice = linw_ref[pl.ds(_IN0 + i * _F, _F), :]
        acc = acc + lax.dot_general(wslice, f, dn, preferred_element_type=jnp.float32)
    out_ref[...] = acc + linb_ref[...]


def _final(parts, root, b, invdeg, x, hs, lin_w, lin_b):
    return pl.pallas_call(
        _final_body,
        out_shape=jax.ShapeDtypeStruct((_F, _N), jnp.float32),
    )(parts, root, b, invdeg, x, *hs, lin_w, lin_b)


_sc_mesh = plsc.VectorSubcoreMesh(core_axis_name="c", subcore_axis_name="s",
                                  num_cores=_NC, num_subcores=_NS)


@functools.partial(
    pl.kernel,
    out_type=jax.ShapeDtypeStruct((_NC, _N, _FP), jnp.float32),
    mesh=_sc_mesh,
    scratch_types=[
        pltpu.VMEM((_NCH, _C), jnp.int32),
        pltpu.VMEM((_NCH, _C), jnp.int32),
        pltpu.VMEM((_NCH, _C), jnp.float32),
        pltpu.VMEM((_NCH, _C), jnp.float32),
        pltpu.VMEM((_C, _FP), jnp.float32),
        pltpu.VMEM((_C, _FP), jnp.float32),
        pltpu.VMEM((_ROWS_PER_SUB, _FP), jnp.float32),
        pltpu.VMEM_SHARED((_N, _FP), jnp.float32),
        pltpu.SemaphoreType.DMA,
        pltpu.SemaphoreType.DMA,
        pltpu.SemaphoreType.DMA,
    ],
)
def _sc_scatter(hall, keys2, dstv2, blo2, bhi2, out, key_v, dst_v, blo_v,
                bhi_v, rows0, rows1, zero_v, acc_sh, sem0, sem1, semi):
    c = lax.axis_index("c")
    s = lax.axis_index("s")
    w = c * _NS + s
    cbase = w * _NCH

    di1 = pltpu.async_copy(keys2.at[pl.ds(cbase, _NCH)], key_v, semi)
    di2 = pltpu.async_copy(dstv2.at[pl.ds(cbase, _NCH)], dst_v, semi)
    di3 = pltpu.async_copy(blo2.at[pl.ds(cbase, _NCH)], blo_v, semi)
    di4 = pltpu.async_copy(bhi2.at[pl.ds(cbase, _NCH)], bhi_v, semi)

    @plsc.parallel_loop(0, _ROWS_PER_SUB)
    def _zrow(r):
        for q in range(_FP // 16):
            zero_v[r, pl.ds(q * 16, 16)] = jnp.zeros((16,), jnp.float32)

    pltpu.sync_copy(zero_v, acc_sh.at[pl.ds(s * _ROWS_PER_SUB, _ROWS_PER_SUB)])
    di1.wait()
    di2.wait()
    di3.wait()
    di4.wait()
    plsc.subcore_barrier()

    def _scale_scatter(i, rows_v):
        @plsc.parallel_loop(0, _C // 16)
        def _scale(jb):
            lchunk = blo_v[i, pl.ds(jb * 16, 16)]
            hchunk = bhi_v[i, pl.ds(jb * 16, 16)]
            for t in range(16):
                lv = lchunk[t]
                hv = hchunk[t]
                r = jb * 16 + t
                for q in range(_F // 16):
                    rows_v[r, pl.ds(q * 16, 16)] = rows_v[r, pl.ds(q * 16, 16)] * lv
                for q in range(_F // 16, _FP // 16):
                    rows_v[r, pl.ds(q * 16, 16)] = rows_v[r, pl.ds(q * 16, 16)] * hv

        pltpu.sync_copy(rows_v, acc_sh.at[dst_v.at[i]], add=True)

    pltpu.async_copy(hall.at[key_v.at[0]], rows0, sem0)

    @pl.loop(0, _NCH, step=2)
    def _chunks(i):
        pltpu.async_copy(hall.at[key_v.at[i + 1]], rows1, sem1)
        pltpu.make_async_copy(hall.at[key_v.at[i]], rows0, sem0).wait()
        _scale_scatter(i, rows0)

        @pl.when(i + 2 < _NCH)
        def _():
            pltpu.async_copy(hall.at[key_v.at[i + 2]], rows0, sem0)

        pltpu.make_async_copy(hall.at[key_v.at[i + 1]], rows1, sem1).wait()
        _scale_scatter(i + 1, rows1)

    plsc.subcore_barrier()
    pltpu.sync_copy(
        acc_sh.at[pl.ds(s * _ROWS_PER_SUB, _ROWS_PER_SUB)],
        out.at[c, pl.ds(s * _ROWS_PER_SUB, _ROWS_PER_SUB)],
    )


def kernel(x, edge_index, edge_attr, w0, root0, b0, w1, root1, b1, w2, root2,
           b2, w3, root3, b3, w4, root4, b4, w5, root5, b5, lin_w, lin_b):
    ws = [w0, w1, w2, w3, w4, w5]
    rs = [root0, root1, root2, root3, root4, root5]
    bs = [b.reshape(1, _F) for b in [b0, b1, b2, b3, b4, b5]]

    keys, dstv, blo, bhi, invdeg = _prep(edge_attr.T, edge_index)
    keys_f = keys.reshape(_T // _C, _C)
    dstv_f = dstv.reshape(_T // _C, _C)
    blo_f = blo.reshape(_T // _C, _C)
    bhi_f = bhi.reshape(_T // _C, _C)

    def _cat(w):
        in_ch = w.shape[1]
        wp = jnp.concatenate(
            [w, jnp.zeros((_KP - _K, in_ch, _F), jnp.float32)], axis=0)
        return wp.transpose(1, 0, 2).reshape(in_ch, _KP * _F)

    wsp = [_cat(w) for w in ws]
    nrows = (_N // 8) * _KT * 8
    hall, root = _mm0(x, wsp[0], rs[0])
    hs = []
    for l in range(1, 6):
        parts = _sc_scatter(hall.reshape(nrows, _FP), keys_f, dstv_f,
                            blo_f, bhi_f)
        hall, root, h = _mm_mid(parts, root, bs[l - 1], invdeg, wsp[l], rs[l])
        hs.append(h)
    parts = _sc_scatter(hall.reshape(nrows, _FP), keys_f, dstv_f, blo_f, bhi_f)
    return _final(parts, root, bs[5], invdeg, x, hs, lin_w,
                  lin_b.reshape(_F, 1))


# speedup vs baseline: 6.6206x; 1.0150x over previous
import functools

import jax
import jax.numpy as jnp
from jax import lax
from jax.experimental import pallas as pl
from jax.experimental.pallas import tpu as pltpu
from jax.experimental.pallas import tpu_sc as plsc

_KS = 5
_DIM = 3
_K = _KS ** _DIM
_N = 2048
_E = 8192
_F = 64
_IN0 = 9
_TAPS = 2 ** _DIM
_T = _E * _TAPS
_NC = 2
_NS = 16
_NW = _NC * _NS
_TPW = _T // _NW
_C = 128
_NCH = _TPW // _C
_ROWS_PER_SUB = _N // _NS
_FP = 128
_KP = _K + 1
_KT = _KP // 2


def _prep_body(attr_t_ref, ei_ref, keys_ref, dstv_ref, blo_ref, bhi_ref,
               invdeg_ref):
    src = ei_ref[0, :]
    dst = ei_ref[1, :]
    p = [attr_t_ref[d, :] * (_KS - 1.0) for d in range(_DIM)]
    lo = [jnp.floor(p[d]) for d in range(_DIM)]
    frac = [p[d] - lo[d] for d in range(_DIM)]
    lo_i = [jnp.clip(lo[d].astype(jnp.int32), 0, _KS - 1) for d in range(_DIM)]
    for b in range(_TAPS):
        basis = jnp.ones((_E,), jnp.float32)
        widx = jnp.zeros((_E,), jnp.int32)
        for d in range(_DIM):
            bit = (b >> d) & 1
            basis = basis * (frac[d] if bit else (1.0 - frac[d]))
            ii = jnp.clip(lo_i[d] + bit, 0, _KS - 1)
            widx = widx * _KS + ii
        keys_ref[b, :] = (widx // 2) * _N + src
        dstv_ref[b, :] = dst
        even = (widx % 2) == 0
        blo_ref[b, :] = jnp.where(even, basis, 0.0)
        bhi_ref[b, :] = jnp.where(even, 0.0, basis)
    acc = jnp.zeros((_N, 1), jnp.float32)
    iota_n = lax.broadcasted_iota(jnp.int32, (_N, 512), 0)
    for c in range(_E // 512):
        dst_c = dst[c * 512:(c + 1) * 512]
        m = (iota_n == dst_c[None, :]).astype(jnp.float32)
        acc = acc + jnp.sum(m, axis=1, keepdims=True)
    invdeg_ref[...] = 1.0 / jnp.maximum(acc, 1.0)


def _prep(attr_t, edge_index):
    return pl.pallas_call(
        _prep_body,
        out_shape=(
            jax.ShapeDtypeStruct((_TAPS, _E), jnp.int32),
            jax.ShapeDtypeStruct((_TAPS, _E), jnp.int32),
            jax.ShapeDtypeStruct((_TAPS, _E), jnp.float32),
            jax.ShapeDtypeStruct((_TAPS, _E), jnp.float32),
            jax.ShapeDtypeStruct((_N, 1), jnp.float32),
        ),
    )(attr_t, edge_index)


def _mm0_body(x_ref, w_ref, r_ref, hall_ref, root_ref):
    k = pl.program_id(0)
    x = x_ref[...]
    hall_ref[0] = jnp.dot(x, w_ref[...], preferred_element_type=jnp.float32)

    @pl.when(k == 0)
    def _():
        root_ref[...] = jnp.dot(x, r_ref[...], preferred_element_type=jnp.float32)


def _mm0(x, wcat, r):
    in_ch = x.shape[1]
    return pl.pallas_call(
        _mm0_body,
        grid=(_KT,),
        in_specs=[
            pl.BlockSpec((_N, in_ch), lambda k: (0, 0)),
            pl.BlockSpec((in_ch, _FP), lambda k: (0, k)),
            pl.BlockSpec((in_ch, _F), lambda k: (0, 0)),
        ],
        out_specs=[
            pl.BlockSpec((1, _N, _FP), lambda k: (k, 0, 0)),
            pl.BlockSpec((_N, _F), lambda k: (0, 0)),
        ],
        out_shape=(
            jax.ShapeDtypeStruct((_KT, _N, _FP), jnp.float32),
            jax.ShapeDtypeStruct((_N, _F), jnp.float32),
        ),
    )(x, wcat, r)


def _mm_mid_body(parts_ref, root_ref, b_ref, invdeg_ref, w_ref, r_ref,
                 hall_ref, rootout_ref, hout_ref, h_scr):
    k = pl.program_id(0)

    @pl.when(k == 0)
    def _():
        msg = (parts_ref[0, :, 0:_F] + parts_ref[0, :, _F:_FP]
               + parts_ref[1, :, 0:_F] + parts_ref[1, :, _F:_FP]) * invdeg_ref[...]
        h = jnp.maximum(msg + root_ref[...] + b_ref[...], 0.0)
        h_scr[...] = h
        hout_ref[...] = h
        rootout_ref[...] = jnp.dot(h, r_ref[...], preferred_element_type=jnp.float32)

    hall_ref[0] = jnp.dot(h_scr[...], w_ref[...], preferred_element_type=jnp.float32)


def _mm_mid(parts, root, b, invdeg, w, r):
    return pl.pallas_call(
        _mm_mid_body,
        grid=(_KT,),
        in_specs=[
            pl.BlockSpec((2, _N, _FP), lambda k: (0, 0, 0)),
            pl.BlockSpec((_N, _F), lambda k: (0, 0)),
            pl.BlockSpec((1, _F), lambda k: (0, 0)),
            pl.BlockSpec((_N, 1), lambda k: (0, 0)),
            pl.BlockSpec((_F, _FP), lambda k: (0, k)),
            pl.BlockSpec((_F, _F), lambda k: (0, 0)),
        ],
        out_specs=[
            pl.BlockSpec((1, _N, _FP), lambda k: (k, 0, 0)),
            pl.BlockSpec((_N, _F), lambda k: (0, 0)),
            pl.BlockSpec((_N, _F), lambda k: (0, 0)),
        ],
        out_shape=(
            jax.ShapeDtypeStruct((_KT, _N, _FP), jnp.float32),
            jax.ShapeDtypeStruct((_N, _F), jnp.float32),
            jax.ShapeDtypeStruct((_N, _F), jnp.float32),
        ),
        scratch_shapes=[pltpu.VMEM((_N, _F), jnp.float32)],
    )(parts, root, b, invdeg, w, r)


def _final_body(parts_ref, root_ref, b_ref, invdeg_ref, x_ref,
                h1, h2, h3, h4, h5, linw_ref, linb_ref, out_ref):
    msg = (parts_ref[0, :, 0:_F] + parts_ref[0, :, _F:_FP]
           + parts_ref[1, :, 0:_F] + parts_ref[1, :, _F:_FP]) * invdeg_ref[...]
    h6 = jnp.maximum(msg + root_ref[...] + b_ref[...], 0.0)
    dn = (((0,), (1,)), ((), ()))
    acc = lax.dot_general(linw_ref[pl.ds(0, _IN0), :], x_ref[...], dn,
                          preferred_element_type=jnp.float32)
    feats = [h1[...], h2[...], h3[...], h4[...], h5[...], h6]
    for i, f in enumerate(feats):
        wslice = linw_ref[pl.ds(_IN0 + i * _F, _F), :]
        acc = acc + lax.dot_general(wslice, f, dn, preferred_element_type=jnp.float32)
    out_ref[...] = acc + linb_ref[...]


def _final(parts, root, b, invdeg, x, hs, lin_w, lin_b):
    return pl.pallas_call(
        _final_body,
        out_shape=jax.ShapeDtypeStruct((_F, _N), jnp.float32),
    )(parts, root, b, invdeg, x, *hs, lin_w, lin_b)


_sc_mesh = plsc.VectorSubcoreMesh(core_axis_name="c", subcore_axis_name="s",
                                  num_cores=_NC, num_subcores=_NS)


@functools.partial(
    pl.kernel,
    out_type=jax.ShapeDtypeStruct((_NC, _N, _FP), jnp.float32),
    mesh=_sc_mesh,
    scratch_types=[
        pltpu.VMEM((_NCH, _C), jnp.int32),
        pltpu.VMEM((_NCH, _C), jnp.int32),
        pltpu.VMEM((_NCH, _C), jnp.float32),
        pltpu.VMEM((_NCH, _C), jnp.float32),
        pltpu.VMEM((_C, _FP), jnp.float32),
        pltpu.VMEM((_C, _FP), jnp.float32),
        pltpu.VMEM((_ROWS_PER_SUB, _FP), jnp.float32),
        pltpu.VMEM_SHARED((_N, _FP), jnp.float32),
        pltpu.SemaphoreType.DMA,
        pltpu.SemaphoreType.DMA,
        pltpu.SemaphoreType.DMA,
    ],
)
def _sc_scatter(hall, keys2, dstv2, blo2, bhi2, out, key_v, dst_v, blo_v,
                bhi_v, rows0, rows1, zero_v, acc_sh, sem0, sem1, semi):
    c = lax.axis_index("c")
    s = lax.axis_index("s")
    w = c * _NS + s
    cbase = w * _NCH

    di1 = pltpu.async_copy(keys2.at[pl.ds(cbase, _NCH)], key_v, semi)
    di2 = pltpu.async_copy(dstv2.at[pl.ds(cbase, _NCH)], dst_v, semi)
    di3 = pltpu.async_copy(blo2.at[pl.ds(cbase, _NCH)], blo_v, semi)
    di4 = pltpu.async_copy(bhi2.at[pl.ds(cbase, _NCH)], bhi_v, semi)

    @plsc.parallel_loop(0, _ROWS_PER_SUB)
    def _zrow(r):
        for q in range(_FP // 16):
            zero_v[r, pl.ds(q * 16, 16)] = jnp.zeros((16,), jnp.float32)

    pltpu.sync_copy(zero_v, acc_sh.at[pl.ds(s * _ROWS_PER_SUB, _ROWS_PER_SUB)])
    di1.wait()
    di2.wait()
    di3.wait()
    di4.wait()
    plsc.subcore_barrier()

    def _scale_scatter(i, rows_v):
        @plsc.parallel_loop(0, _C // 16)
        def _scale(jb):
            lchunk = blo_v[i, pl.ds(jb * 16, 16)]
            hchunk = bhi_v[i, pl.ds(jb * 16, 16)]
            for t in range(16):
                lv = lchunk[t]
                hv = hchunk[t]
                r = jb * 16 + t
                for q in range(_F // 16):
                    rows_v[r, pl.ds(q * 16, 16)] = rows_v[r, pl.ds(q * 16, 16)] * lv
                for q in range(_F // 16, _FP // 16):
                    rows_v[r, pl.ds(q * 16, 16)] = rows_v[r, pl.ds(q * 16, 16)] * hv

        pltpu.sync_copy(rows_v, acc_sh.at[dst_v.at[i]], add=True)

    pltpu.async_copy(hall.at[key_v.at[0]], rows0, sem0)

    @pl.loop(0, _NCH, step=2)
    def _chunks(i):
        pltpu.async_copy(hall.at[key_v.at[i + 1]], rows1, sem1)
        pltpu.make_async_copy(hall.at[key_v.at[i]], rows0, sem0).wait()
        _scale_scatter(i, rows0)

        @pl.when(i + 2 < _NCH)
        def _():
            pltpu.async_copy(hall.at[key_v.at[i + 2]], rows0, sem0)

        pltpu.make_async_copy(hall.at[key_v.at[i + 1]], rows1, sem1).wait()
        _scale_scatter(i + 1, rows1)

    plsc.subcore_barrier()
    pltpu.sync_copy(
        acc_sh.at[pl.ds(s * _ROWS_PER_SUB, _ROWS_PER_SUB)],
        out.at[c, pl.ds(s * _ROWS_PER_SUB, _ROWS_PER_SUB)],
    )


def kernel(x, edge_index, edge_attr, w0, root0, b0, w1, root1, b1, w2, root2,
           b2, w3, root3, b3, w4, root4, b4, w5, root5, b5, lin_w, lin_b):
    ws = [w0, w1, w2, w3, w4, w5]
    rs = [root0, root1, root2, root3, root4, root5]
    bs = [b.reshape(1, _F) for b in [b0, b1, b2, b3, b4, b5]]

    keys, dstv, blo, bhi, invdeg = _prep(edge_attr.T, edge_index)
    keys_f = keys.reshape(_T // _C, _C)
    dstv_f = dstv.reshape(_T // _C, _C)
    blo_f = blo.reshape(_T // _C, _C)
    bhi_f = bhi.reshape(_T // _C, _C)

    def _cat(w):
        in_ch = w.shape[1]
        wp = jnp.concatenate(
            [w, jnp.zeros((_KP - _K, in_ch, _F), jnp.float32)], axis=0)
        return wp.transpose(1, 0, 2).reshape(in_ch, _KP * _F)

    wsp = [_cat(w) for w in ws]
    nrows = _KT * _N
    hall, root = _mm0(x, wsp[0], rs[0])
    hs = []
    for l in range(1, 6):
        parts = _sc_scatter(hall.reshape(nrows, _FP), keys_f, dstv_f,
                            blo_f, bhi_f)
        hall, root, h = _mm_mid(parts, root, bs[l - 1], invdeg, wsp[l], rs[l])
        hs.append(h)
    parts = _sc_scatter(hall.reshape(nrows, _FP), keys_f, dstv_f, blo_f, bhi_f)
    return _final(parts, root, bs[5], invdeg, x, hs, lin_w,
                  lin_b.reshape(_F, 1))


# speedup vs baseline: 9.4432x; 1.4263x over previous
import functools

import jax
import jax.numpy as jnp
from jax import lax
from jax.experimental import pallas as pl
from jax.experimental.pallas import tpu as pltpu
from jax.experimental.pallas import tpu_sc as plsc

_KS = 5
_DIM = 3
_K = _KS ** _DIM
_N = 2048
_E = 8192
_F = 64
_IN0 = 9
_TAPS = 2 ** _DIM
_T = _E * _TAPS
_NC = 2
_NS = 16
_NW = _NC * _NS
_TPW = _T // _NW
_C = 128
_NCH = _TPW // _C
_ROWS_PER_SUB = _N // _NS
_FP = 128
_KP = _K + 1
_KT = _KP // 2


def _prep_body(attr_t_ref, ei_ref, keys_ref, dstv_ref, blo_ref, bhi_ref,
               invdeg_ref):
    src = ei_ref[0, :]
    dst = ei_ref[1, :]
    p = [attr_t_ref[d, :] * (_KS - 1.0) for d in range(_DIM)]
    lo = [jnp.floor(p[d]) for d in range(_DIM)]
    frac = [p[d] - lo[d] for d in range(_DIM)]
    lo_i = [jnp.clip(lo[d].astype(jnp.int32), 0, _KS - 1) for d in range(_DIM)]
    for b in range(_TAPS):
        basis = jnp.ones((_E,), jnp.float32)
        widx = jnp.zeros((_E,), jnp.int32)
        for d in range(_DIM):
            bit = (b >> d) & 1
            basis = basis * (frac[d] if bit else (1.0 - frac[d]))
            ii = jnp.clip(lo_i[d] + bit, 0, _KS - 1)
            widx = widx * _KS + ii
        keys_ref[b, :] = (widx // 2) * _N + src
        dstv_ref[b, :] = dst
        even = (widx % 2) == 0
        blo_ref[b, :] = jnp.where(even, basis, 0.0)
        bhi_ref[b, :] = jnp.where(even, 0.0, basis)
    acc = jnp.zeros((_N, 1), jnp.float32)
    iota_n = lax.broadcasted_iota(jnp.int32, (_N, 512), 0)
    for c in range(_E // 512):
        dst_c = dst[c * 512:(c + 1) * 512]
        m = (iota_n == dst_c[None, :]).astype(jnp.float32)
        acc = acc + jnp.sum(m, axis=1, keepdims=True)
    invdeg_ref[...] = 1.0 / jnp.maximum(acc, 1.0)


def _prep(attr_t, edge_index):
    return pl.pallas_call(
        _prep_body,
        out_shape=(
            jax.ShapeDtypeStruct((_TAPS, _E), jnp.int32),
            jax.ShapeDtypeStruct((_TAPS, _E), jnp.int32),
            jax.ShapeDtypeStruct((_TAPS, _E), jnp.float32),
            jax.ShapeDtypeStruct((_TAPS, _E), jnp.float32),
            jax.ShapeDtypeStruct((_N, 1), jnp.float32),
        ),
    )(attr_t, edge_index)


_KB = 7
_KG = _KT // _KB


def _mm0_body(x_ref, w_ref, r_ref, hall_ref, root_ref):
    k = pl.program_id(0)
    x = x_ref[...]
    for j in range(_KB):
        hall_ref[j] = jnp.dot(x, w_ref[:, j * _FP:(j + 1) * _FP],
                              preferred_element_type=jnp.float32)

    @pl.when(k == 0)
    def _():
        root_ref[...] = jnp.dot(x, r_ref[...], preferred_element_type=jnp.float32)


def _mm0(x, wcat, r):
    in_ch = x.shape[1]
    return pl.pallas_call(
        _mm0_body,
        grid=(_KG,),
        in_specs=[
            pl.BlockSpec((_N, in_ch), lambda k: (0, 0)),
            pl.BlockSpec((in_ch, _KB * _FP), lambda k: (0, k)),
            pl.BlockSpec((in_ch, _F), lambda k: (0, 0)),
        ],
        out_specs=[
            pl.BlockSpec((_KB, _N, _FP), lambda k: (k, 0, 0)),
            pl.BlockSpec((_N, _F), lambda k: (0, 0)),
        ],
        out_shape=(
            jax.ShapeDtypeStruct((_KT, _N, _FP), jnp.float32),
            jax.ShapeDtypeStruct((_N, _F), jnp.float32),
        ),
    )(x, wcat, r)


def _mm_mid_body(parts_ref, root_ref, b_ref, invdeg_ref, w_ref, r_ref,
                 hall_ref, rootout_ref, hout_ref, h_scr):
    k = pl.program_id(0)

    @pl.when(k == 0)
    def _():
        msg = (parts_ref[0] + parts_ref[1]) * invdeg_ref[...]
        h = jnp.maximum(msg + root_ref[...] + b_ref[...], 0.0)
        h_scr[...] = h
        hout_ref[...] = h
        rootout_ref[...] = jnp.dot(h, r_ref[...], preferred_element_type=jnp.float32)

    for j in range(_KB):
        hall_ref[j] = jnp.dot(h_scr[...], w_ref[:, j * _FP:(j + 1) * _FP],
                              preferred_element_type=jnp.float32)


def _mm_mid(parts, root, b, invdeg, w, r):
    return pl.pallas_call(
        _mm_mid_body,
        grid=(_KG,),
        in_specs=[
            pl.BlockSpec((2, _N, _F), lambda k: (0, 0, 0)),
            pl.BlockSpec((_N, _F), lambda k: (0, 0)),
            pl.BlockSpec((1, _F), lambda k: (0, 0)),
            pl.BlockSpec((_N, 1), lambda k: (0, 0)),
            pl.BlockSpec((_F, _KB * _FP), lambda k: (0, k)),
            pl.BlockSpec((_F, _F), lambda k: (0, 0)),
        ],
        out_specs=[
            pl.BlockSpec((_KB, _N, _FP), lambda k: (k, 0, 0)),
            pl.BlockSpec((_N, _F), lambda k: (0, 0)),
            pl.BlockSpec((_N, _F), lambda k: (0, 0)),
        ],
        out_shape=(
            jax.ShapeDtypeStruct((_KT, _N, _FP), jnp.float32),
            jax.ShapeDtypeStruct((_N, _F), jnp.float32),
            jax.ShapeDtypeStruct((_N, _F), jnp.float32),
        ),
        scratch_shapes=[pltpu.VMEM((_N, _F), jnp.float32)],
    )(parts, root, b, invdeg, w, r)


def _final_body(parts_ref, root_ref, b_ref, invdeg_ref, x_ref,
                h1, h2, h3, h4, h5, linw_ref, linb_ref, out_ref):
    msg = (parts_ref[0] + parts_ref[1]) * invdeg_ref[...]
    h6 = jnp.maximum(msg + root_ref[...] + b_ref[...], 0.0)
    dn = (((0,), (1,)), ((), ()))
    acc = lax.dot_general(linw_ref[pl.ds(0, _IN0), :], x_ref[...], dn,
                          preferred_element_type=jnp.float32)
    feats = [h1[...], h2[...], h3[...], h4[...], h5[...], h6]
    for i, f in enumerate(feats):
        wslice = linw_ref[pl.ds(_IN0 + i * _F, _F), :]
        acc = acc + lax.dot_general(wslice, f, dn, preferred_element_type=jnp.float32)
    out_ref[...] = acc + linb_ref[...]


def _final(parts, root, b, invdeg, x, hs, lin_w, lin_b):
    return pl.pallas_call(
        _final_body,
        out_shape=jax.ShapeDtypeStruct((_F, _N), jnp.float32),
    )(parts, root, b, invdeg, x, *hs, lin_w, lin_b)


_sc_mesh = plsc.VectorSubcoreMesh(core_axis_name="c", subcore_axis_name="s",
                                  num_cores=_NC, num_subcores=_NS)


@functools.partial(
    pl.kernel,
    out_type=jax.ShapeDtypeStruct((_NC, _N, _F), jnp.float32),
    mesh=_sc_mesh,
    scratch_types=[
        pltpu.VMEM((_NCH, _C), jnp.int32),
        pltpu.VMEM((_NCH, _C), jnp.int32),
        pltpu.VMEM((_NCH, _C), jnp.float32),
        pltpu.VMEM((_NCH, _C), jnp.float32),
        pltpu.VMEM((_C, _FP), jnp.float32),
        pltpu.VMEM((_C, _FP), jnp.float32),
        pltpu.VMEM((_ROWS_PER_SUB, _FP), jnp.float32),
        pltpu.VMEM((_ROWS_PER_SUB, _F), jnp.float32),
        pltpu.VMEM_SHARED((_N, _FP), jnp.float32),
        pltpu.SemaphoreType.DMA,
        pltpu.SemaphoreType.DMA,
        pltpu.SemaphoreType.DMA,
    ],
)
def _sc_scatter(hall, keys2, dstv2, blo2, bhi2, out, key_v, dst_v, blo_v,
                bhi_v, rows0, rows1, zero_v, fold_v, acc_sh, sem0, sem1, semi):
    c = lax.axis_index("c")
    s = lax.axis_index("s")
    w = c * _NS + s
    cbase = w * _NCH

    di1 = pltpu.async_copy(keys2.at[pl.ds(cbase, _NCH)], key_v, semi)
    di2 = pltpu.async_copy(dstv2.at[pl.ds(cbase, _NCH)], dst_v, semi)
    di3 = pltpu.async_copy(blo2.at[pl.ds(cbase, _NCH)], blo_v, semi)
    di4 = pltpu.async_copy(bhi2.at[pl.ds(cbase, _NCH)], bhi_v, semi)

    @plsc.parallel_loop(0, _ROWS_PER_SUB)
    def _zrow(r):
        for q in range(_FP // 16):
            zero_v[r, pl.ds(q * 16, 16)] = jnp.zeros((16,), jnp.float32)

    pltpu.sync_copy(zero_v, acc_sh.at[pl.ds(s * _ROWS_PER_SUB, _ROWS_PER_SUB)])
    di1.wait()
    di2.wait()
    di3.wait()
    di4.wait()
    plsc.subcore_barrier()

    def _scale_scatter(i, rows_v):
        @plsc.parallel_loop(0, _C // 16)
        def _scale(jb):
            lchunk = blo_v[i, pl.ds(jb * 16, 16)]
            hchunk = bhi_v[i, pl.ds(jb * 16, 16)]
            for t in range(16):
                lv = lchunk[t]
                hv = hchunk[t]
                r = jb * 16 + t
                for q in range(_F // 16):
                    rows_v[r, pl.ds(q * 16, 16)] = rows_v[r, pl.ds(q * 16, 16)] * lv
                for q in range(_F // 16, _FP // 16):
                    rows_v[r, pl.ds(q * 16, 16)] = rows_v[r, pl.ds(q * 16, 16)] * hv

        pltpu.sync_copy(rows_v, acc_sh.at[dst_v.at[i]], add=True)

    pltpu.async_copy(hall.at[key_v.at[0]], rows0, sem0)

    @pl.loop(0, _NCH, step=2)
    def _chunks(i):
        pltpu.async_copy(hall.at[key_v.at[i + 1]], rows1, sem1)
        pltpu.make_async_copy(hall.at[key_v.at[i]], rows0, sem0).wait()
        _scale_scatter(i, rows0)

        @pl.when(i + 2 < _NCH)
        def _():
            pltpu.async_copy(hall.at[key_v.at[i + 2]], rows0, sem0)

        pltpu.make_async_copy(hall.at[key_v.at[i + 1]], rows1, sem1).wait()
        _scale_scatter(i + 1, rows1)

    plsc.subcore_barrier()
    rbase = s * _ROWS_PER_SUB
    pltpu.sync_copy(acc_sh.at[pl.ds(rbase, _ROWS_PER_SUB)], zero_v)

    @plsc.parallel_loop(0, _ROWS_PER_SUB)
    def _fold(r):
        for q in range(_F // 16):
            fold_v[r, pl.ds(q * 16, 16)] = (
                zero_v[r, pl.ds(q * 16, 16)]
                + zero_v[r, pl.ds(_F + q * 16, 16)])

    pltpu.sync_copy(fold_v, out.at[c, pl.ds(rbase, _ROWS_PER_SUB)])


def kernel(x, edge_index, edge_attr, w0, root0, b0, w1, root1, b1, w2, root2,
           b2, w3, root3, b3, w4, root4, b4, w5, root5, b5, lin_w, lin_b):
    ws = [w0, w1, w2, w3, w4, w5]
    rs = [root0, root1, root2, root3, root4, root5]
    bs = [b.reshape(1, _F) for b in [b0, b1, b2, b3, b4, b5]]

    keys, dstv, blo, bhi, invdeg = _prep(edge_attr.T, edge_index)
    keys_f = keys.reshape(_T // _C, _C)
    dstv_f = dstv.reshape(_T // _C, _C)
    blo_f = blo.reshape(_T // _C, _C)
    bhi_f = bhi.reshape(_T // _C, _C)

    def _cat(w):
        in_ch = w.shape[1]
        wp = jnp.concatenate(
            [w, jnp.zeros((_KP - _K, in_ch, _F), jnp.float32)], axis=0)
        return wp.transpose(1, 0, 2).reshape(in_ch, _KP * _F)

    wsp = [_cat(w) for w in ws]
    nrows = _KT * _N
    hall, root = _mm0(x, wsp[0], rs[0])
    hs = []
    for l in range(1, 6):
        parts = _sc_scatter(hall.reshape(nrows, _FP), keys_f, dstv_f,
                            blo_f, bhi_f)
        hall, root, h = _mm_mid(parts, root, bs[l - 1], invdeg, wsp[l], rs[l])
        hs.append(h)
    parts = _sc_scatter(hall.reshape(nrows, _FP), keys_f, dstv_f, blo_f, bhi_f)
    return _final(parts, root, bs[5], invdeg, x, hs, lin_w,
                  lin_b.reshape(_F, 1))
